# baseline (speedup 1.0000x reference)
import functools

import jax
import jax.numpy as jnp
from jax import lax
from jax.experimental import pallas as pl
from jax.experimental.pallas import tpu as pltpu
from jax.experimental.pallas import tpu_sc as plsc

B = 16384
D = 64
A = 16
M0 = 10000
M1 = 100

_info = plsc.get_sparse_core_info()
NC, NS, L = _info.num_cores, _info.num_subcores, _info.num_lanes
NW = NC * NS
BPW = B // NW
CHUNK = 128
NCHUNK = BPW // CHUNK
GPC = CHUNK // L


def _sc_gather(state, table):
    mesh = plsc.VectorSubcoreMesh(core_axis_name="c", subcore_axis_name="s")

    @functools.partial(
        pl.kernel,
        mesh=mesh,
        compiler_params=pltpu.CompilerParams(
            needs_layout_passes=False, use_tc_tiling_on_sc=False
        ),
        out_type=jax.ShapeDtypeStruct((B, D), jnp.float32),
        scratch_types=[
            pltpu.VMEM((BPW * 3,), jnp.int32),
            pltpu.VMEM((NCHUNK, CHUNK), jnp.int32),
            pltpu.VMEM((BPW, D), jnp.float32),
            pltpu.SemaphoreType.DMA,
        ],
    )
    def k(state_hbm, table_hbm, out_hbm, state_v, ids_v, rows_v, sem):
        wid = lax.axis_index("s") * NC + lax.axis_index("c")
        base = wid * BPW
        pltpu.sync_copy(state_hbm.at[pl.ds(base * 3, BPW * 3)], state_v)
        lanes3 = lax.iota(jnp.int32, L) * 3
        for g in range(BPW // L):
            r3 = lanes3 + g * (L * 3)
            s0 = plsc.load_gather(state_v, [r3])
            s1 = plsc.load_gather(state_v, [r3 + 1])
            s2 = plsc.load_gather(state_v, [r3 + 2])
            ids_v[g // GPC, pl.ds((g % GPC) * L, L)] = s0 * M0 + s1 * M1 + s2
        copies = [
            pltpu.async_copy(
                table_hbm.at[ids_v.at[cg]],
                rows_v.at[pl.ds(cg * CHUNK, CHUNK)],
                sem,
            )
            for cg in range(NCHUNK)
        ]
        for c in copies:
            c.wait()
        pltpu.sync_copy(rows_v, out_hbm.at[pl.ds(base, BPW)])

    return k(state.reshape(-1), table)


def _mm_body(emb_ref, w_ref, b_ref, out_ref):
    out_ref[...] = (
        jnp.dot(emb_ref[...], w_ref[...], preferred_element_type=jnp.float32)
        + b_ref[...]
    )


def _tc_matmul(emb, W, b2):
    BM = 2048
    return pl.pallas_call(
        _mm_body,
        grid=(B // BM,),
        in_specs=[
            pl.BlockSpec((BM, D), lambda i: (i, 0)),
            pl.BlockSpec((D, A), lambda i: (0, 0)),
            pl.BlockSpec((1, A), lambda i: (0, 0)),
        ],
        out_specs=pl.BlockSpec((BM, A), lambda i: (i, 0)),
        out_shape=jax.ShapeDtypeStruct((B, A), jnp.float32),
    )(emb, W, b2)


def kernel(state, table, W, b):
    emb = _sc_gather(state, table)
    return _tc_matmul(emb, W, b.reshape(1, A))
